# baseline (speedup 1.0000x reference)
import jax
import jax.numpy as jnp
from jax import lax
from jax.experimental import pallas as pl
from jax.experimental.pallas import tpu as pltpu
from jax.experimental.pallas import tpu_sc as plsc

DIM = 128
BATCH = 16384

NC = 2
NS = 16
L = 16
NW = NC * NS
BPW = BATCH // NW
CHUNK = 128
NCHUNK = BPW // CHUNK
GROUPS = CHUNK // L


def _gmf_body(user_hbm, item_hbm, utab_hbm, itab_hbm, out_hbm,
              u_idx, i_idx, u_rows0, i_rows0, u_rows1, i_rows1, o_v,
              sem_u0, sem_i0, sem_u1, sem_i1):
    wid = lax.axis_index("s") * NC + lax.axis_index("c")
    base = wid * BPW

    pltpu.sync_copy(user_hbm.at[pl.ds(base, BPW)], u_idx)
    pltpu.sync_copy(item_hbm.at[pl.ds(base, BPW)], i_idx)

    iota = lax.broadcasted_iota(jnp.int32, (L,), 0)

    bufs = [(u_rows0, i_rows0, sem_u0, sem_i0),
            (u_rows1, i_rows1, sem_u1, sem_i1)]

    def issue(c):
        ub, ib, su, si = bufs[c % 2]
        cu = pltpu.async_copy(utab_hbm.at[u_idx.at[pl.ds(c * CHUNK, CHUNK)]],
                              ub, su)
        ci = pltpu.async_copy(itab_hbm.at[i_idx.at[pl.ds(c * CHUNK, CHUNK)]],
                              ib, si)
        return cu, ci

    inflight = issue(0)
    for c in range(NCHUNK):
        if c + 1 < NCHUNK:
            nxt = issue(c + 1)
        inflight[0].wait()
        inflight[1].wait()
        ub, ib, _, _ = bufs[c % 2]

        def group_body(g, _, ub=ub, ib=ib, c=c):
            rvec = iota + g * L
            a0 = jnp.zeros((L,), jnp.float32)
            a1 = jnp.zeros((L,), jnp.float32)
            a2 = jnp.zeros((L,), jnp.float32)
            a3 = jnp.zeros((L,), jnp.float32)
            accs = [a0, a1, a2, a3]
            for d in range(DIM):
                dvec = jnp.full((L,), d, jnp.int32)
                uv = plsc.load_gather(ub, [rvec, dvec])
                iv = plsc.load_gather(ib, [rvec, dvec])
                accs[d & 3] = accs[d & 3] + uv * iv
            tot = (accs[0] + accs[1]) + (accs[2] + accs[3])
            o_v[pl.ds(c * CHUNK + g * L, L)] = 1.0 / (1.0 + jnp.exp(-tot))
            return 0

        lax.fori_loop(0, GROUPS, group_body, 0)
        inflight = nxt if c + 1 < NCHUNK else inflight

    pltpu.sync_copy(o_v, out_hbm.at[pl.ds(base, BPW)])


@jax.jit
def _gmf(user1d, item1d, user_table, item_table):
    mesh = plsc.VectorSubcoreMesh(core_axis_name="c", subcore_axis_name="s")
    kern = pl.kernel(
        _gmf_body,
        mesh=mesh,
        out_type=jax.ShapeDtypeStruct((BATCH,), jnp.float32),
        compiler_params=pltpu.CompilerParams(needs_layout_passes=False),
        scratch_types=[
            pltpu.VMEM((BPW,), jnp.int32),
            pltpu.VMEM((BPW,), jnp.int32),
            pltpu.VMEM((CHUNK, DIM), jnp.float32),
            pltpu.VMEM((CHUNK, DIM), jnp.float32),
            pltpu.VMEM((CHUNK, DIM), jnp.float32),
            pltpu.VMEM((CHUNK, DIM), jnp.float32),
            pltpu.VMEM((BPW,), jnp.float32),
            pltpu.SemaphoreType.DMA,
            pltpu.SemaphoreType.DMA,
            pltpu.SemaphoreType.DMA,
            pltpu.SemaphoreType.DMA,
        ],
    )
    return kern(user1d, item1d, user_table, item_table)


def kernel(user, item, user_table, item_table):
    return _gmf(user.astype(jnp.int32), item.astype(jnp.int32),
                user_table, item_table)


# speedup vs baseline: 2.6628x; 2.6628x over previous
import jax
import jax.numpy as jnp
from jax import lax
from jax.experimental import pallas as pl
from jax.experimental.pallas import tpu as pltpu
from jax.experimental.pallas import tpu_sc as plsc

DIM = 128
BATCH = 16384

NC = 2
NS = 16
L = 16
NW = NC * NS
BPW = BATCH // NW
CHUNK = 128
NCHUNK = BPW // CHUNK
GROUPS = CHUNK // L


def _gmf_body(user_hbm, item_hbm, utab_hbm, itab_hbm, out_hbm,
              u_idx, i_idx, u_rows0, i_rows0, u_rows1, i_rows1, o_v, accs,
              sem_u0, sem_i0, sem_u1, sem_i1):
    wid = lax.axis_index("s") * NC + lax.axis_index("c")
    base = wid * BPW

    pltpu.sync_copy(user_hbm.at[pl.ds(base, BPW)], u_idx)
    pltpu.sync_copy(item_hbm.at[pl.ds(base, BPW)], i_idx)

    iota = lax.broadcasted_iota(jnp.int32, (L,), 0)

    bufs = [(u_rows0, i_rows0, sem_u0, sem_i0),
            (u_rows1, i_rows1, sem_u1, sem_i1)]

    def issue(c):
        ub, ib, su, si = bufs[c % 2]
        cu = pltpu.async_copy(utab_hbm.at[u_idx.at[pl.ds(c * CHUNK, CHUNK)]],
                              ub, su)
        ci = pltpu.async_copy(itab_hbm.at[i_idx.at[pl.ds(c * CHUNK, CHUNK)]],
                              ib, si)
        return cu, ci

    inflight = issue(0)
    for c in range(NCHUNK):
        if c + 1 < NCHUNK:
            nxt = issue(c + 1)
        inflight[0].wait()
        inflight[1].wait()
        ub, ib, _, _ = bufs[c % 2]

        def group_body(g, _, ub=ub, ib=ib, c=c):
            for rr in range(L):
                r = g * L + rr
                p0 = ub[r, pl.ds(0 * L, L)] * ib[r, pl.ds(0 * L, L)]
                p1 = ub[r, pl.ds(1 * L, L)] * ib[r, pl.ds(1 * L, L)]
                p2 = ub[r, pl.ds(2 * L, L)] * ib[r, pl.ds(2 * L, L)]
                p3 = ub[r, pl.ds(3 * L, L)] * ib[r, pl.ds(3 * L, L)]
                p4 = ub[r, pl.ds(4 * L, L)] * ib[r, pl.ds(4 * L, L)]
                p5 = ub[r, pl.ds(5 * L, L)] * ib[r, pl.ds(5 * L, L)]
                p6 = ub[r, pl.ds(6 * L, L)] * ib[r, pl.ds(6 * L, L)]
                p7 = ub[r, pl.ds(7 * L, L)] * ib[r, pl.ds(7 * L, L)]
                s = ((p0 + p1) + (p2 + p3)) + ((p4 + p5) + (p6 + p7))
                accs[rr, pl.ds(0, L)] = s
            t0 = plsc.load_gather(accs, [iota, jnp.zeros((L,), jnp.int32)])
            t1 = plsc.load_gather(accs, [iota, jnp.zeros((L,), jnp.int32) + 1])
            for j in range(2, L, 2):
                t0 = t0 + plsc.load_gather(
                    accs, [iota, jnp.zeros((L,), jnp.int32) + j])
                t1 = t1 + plsc.load_gather(
                    accs, [iota, jnp.zeros((L,), jnp.int32) + j + 1])
            tot = t0 + t1
            o_v[pl.ds(c * CHUNK + g * L, L)] = 1.0 / (1.0 + jnp.exp(-tot))
            return 0

        lax.fori_loop(0, GROUPS, group_body, 0)
        inflight = nxt if c + 1 < NCHUNK else inflight

    pltpu.sync_copy(o_v, out_hbm.at[pl.ds(base, BPW)])


@jax.jit
def _gmf(user1d, item1d, user_table, item_table):
    mesh = plsc.VectorSubcoreMesh(core_axis_name="c", subcore_axis_name="s")
    kern = pl.kernel(
        _gmf_body,
        mesh=mesh,
        out_type=jax.ShapeDtypeStruct((BATCH,), jnp.float32),
        compiler_params=pltpu.CompilerParams(needs_layout_passes=False),
        scratch_types=[
            pltpu.VMEM((BPW,), jnp.int32),
            pltpu.VMEM((BPW,), jnp.int32),
            pltpu.VMEM((CHUNK, DIM), jnp.float32),
            pltpu.VMEM((CHUNK, DIM), jnp.float32),
            pltpu.VMEM((CHUNK, DIM), jnp.float32),
            pltpu.VMEM((CHUNK, DIM), jnp.float32),
            pltpu.VMEM((BPW,), jnp.float32),
            pltpu.VMEM((L, L + 1), jnp.float32),
            pltpu.SemaphoreType.DMA,
            pltpu.SemaphoreType.DMA,
            pltpu.SemaphoreType.DMA,
            pltpu.SemaphoreType.DMA,
        ],
    )
    return kern(user1d, item1d, user_table, item_table)


def kernel(user, item, user_table, item_table):
    return _gmf(user.astype(jnp.int32), item.astype(jnp.int32),
                user_table, item_table)


# speedup vs baseline: 2.9307x; 1.1006x over previous
import jax
import jax.numpy as jnp
from jax import lax
from jax.experimental import pallas as pl
from jax.experimental.pallas import tpu as pltpu
from jax.experimental.pallas import tpu_sc as plsc

DIM = 128
BATCH = 16384

NC = 2
NS = 16
L = 16
NW = NC * NS
BPW = BATCH // NW
CHUNK = 128
NCHUNK = BPW // CHUNK
GROUPS = CHUNK // L


def _gmf_body(user_hbm, item_hbm, utab_hbm, itab_hbm, out_hbm,
              u_idx, i_idx, u_buf, i_buf, o_v, accs, sem_u, sem_i):
    wid = lax.axis_index("s") * NC + lax.axis_index("c")
    base = wid * BPW

    pltpu.sync_copy(user_hbm.at[pl.ds(base, BPW)], u_idx)
    pltpu.sync_copy(item_hbm.at[pl.ds(base, BPW)], i_idx)

    iota = lax.broadcasted_iota(jnp.int32, (L,), 0)

    def issue(c):
        boff = (c % 2) * CHUNK
        pltpu.async_copy(utab_hbm.at[u_idx.at[pl.ds(c * CHUNK, CHUNK)]],
                         u_buf.at[pl.ds(boff, CHUNK)], sem_u)
        pltpu.async_copy(itab_hbm.at[i_idx.at[pl.ds(c * CHUNK, CHUNK)]],
                         i_buf.at[pl.ds(boff, CHUNK)], sem_i)

    issue(0)
    issue(1)

    def chunk_body(c, _):
        boff = (c % 2) * CHUNK
        pltpu.make_async_copy(utab_hbm.at[u_idx.at[pl.ds(0, CHUNK)]],
                              u_buf.at[pl.ds(boff, CHUNK)], sem_u).wait()
        pltpu.make_async_copy(itab_hbm.at[i_idx.at[pl.ds(0, CHUNK)]],
                              i_buf.at[pl.ds(boff, CHUNK)], sem_i).wait()

        def group_body(g, _):
            row = boff + g * L
            for rr in range(L):
                r = row + rr
                p0 = u_buf[r, pl.ds(0 * L, L)] * i_buf[r, pl.ds(0 * L, L)]
                p1 = u_buf[r, pl.ds(1 * L, L)] * i_buf[r, pl.ds(1 * L, L)]
                p2 = u_buf[r, pl.ds(2 * L, L)] * i_buf[r, pl.ds(2 * L, L)]
                p3 = u_buf[r, pl.ds(3 * L, L)] * i_buf[r, pl.ds(3 * L, L)]
                p4 = u_buf[r, pl.ds(4 * L, L)] * i_buf[r, pl.ds(4 * L, L)]
                p5 = u_buf[r, pl.ds(5 * L, L)] * i_buf[r, pl.ds(5 * L, L)]
                p6 = u_buf[r, pl.ds(6 * L, L)] * i_buf[r, pl.ds(6 * L, L)]
                p7 = u_buf[r, pl.ds(7 * L, L)] * i_buf[r, pl.ds(7 * L, L)]
                s = ((p0 + p1) + (p2 + p3)) + ((p4 + p5) + (p6 + p7))
                accs[rr, pl.ds(0, L)] = s
            t0 = plsc.load_gather(accs, [iota, jnp.zeros((L,), jnp.int32)])
            t1 = plsc.load_gather(accs, [iota, jnp.zeros((L,), jnp.int32) + 1])
            for j in range(2, L, 2):
                t0 = t0 + plsc.load_gather(
                    accs, [iota, jnp.zeros((L,), jnp.int32) + j])
                t1 = t1 + plsc.load_gather(
                    accs, [iota, jnp.zeros((L,), jnp.int32) + j + 1])
            tot = t0 + t1
            o_v[pl.ds(c * CHUNK + g * L, L)] = 1.0 / (1.0 + jnp.exp(-tot))
            return 0

        lax.fori_loop(0, GROUPS, group_body, 0)

        @pl.when(c + 2 < NCHUNK)
        def _():
            pltpu.async_copy(
                utab_hbm.at[u_idx.at[pl.ds((c + 2) * CHUNK, CHUNK)]],
                u_buf.at[pl.ds(boff, CHUNK)], sem_u)
            pltpu.async_copy(
                itab_hbm.at[i_idx.at[pl.ds((c + 2) * CHUNK, CHUNK)]],
                i_buf.at[pl.ds(boff, CHUNK)], sem_i)

        return 0

    lax.fori_loop(0, NCHUNK, chunk_body, 0)

    pltpu.sync_copy(o_v, out_hbm.at[pl.ds(base, BPW)])


@jax.jit
def _gmf(user1d, item1d, user_table, item_table):
    mesh = plsc.VectorSubcoreMesh(core_axis_name="c", subcore_axis_name="s")
    kern = pl.kernel(
        _gmf_body,
        mesh=mesh,
        out_type=jax.ShapeDtypeStruct((BATCH,), jnp.float32),
        compiler_params=pltpu.CompilerParams(needs_layout_passes=False),
        scratch_types=[
            pltpu.VMEM((BPW,), jnp.int32),
            pltpu.VMEM((BPW,), jnp.int32),
            pltpu.VMEM((2 * CHUNK, DIM), jnp.float32),
            pltpu.VMEM((2 * CHUNK, DIM), jnp.float32),
            pltpu.VMEM((BPW,), jnp.float32),
            pltpu.VMEM((L, L + 1), jnp.float32),
            pltpu.SemaphoreType.DMA,
            pltpu.SemaphoreType.DMA,
        ],
    )
    return kern(user1d, item1d, user_table, item_table)


def kernel(user, item, user_table, item_table):
    return _gmf(user.astype(jnp.int32), item.astype(jnp.int32),
                user_table, item_table)


# speedup vs baseline: 3.3040x; 1.1274x over previous
import jax
import jax.numpy as jnp
from jax import lax
from jax.experimental import pallas as pl
from jax.experimental.pallas import tpu as pltpu
from jax.experimental.pallas import tpu_sc as plsc

DIM = 128
BATCH = 16384

NC = 2
NS = 16
L = 16
NW = NC * NS
BPW = BATCH // NW
CHUNK = 128
NCHUNK = BPW // CHUNK
GROUPS = CHUNK // L


def _gmf_body(user_hbm, item_hbm, utab_hbm, itab_hbm, out_hbm,
              u_idx, i_idx, u_buf, i_buf, o_v, accs, sem_u, sem_i):
    wid = lax.axis_index("s") * NC + lax.axis_index("c")
    base = wid * BPW

    pltpu.sync_copy(user_hbm.at[pl.ds(base, BPW)], u_idx)
    pltpu.sync_copy(item_hbm.at[pl.ds(base, BPW)], i_idx)

    iota = lax.broadcasted_iota(jnp.int32, (L,), 0)

    def issue(c):
        boff = (c % 2) * CHUNK
        pltpu.async_copy(utab_hbm.at[u_idx.at[pl.ds(c * CHUNK, CHUNK)]],
                         u_buf.at[pl.ds(boff, CHUNK)], sem_u)
        pltpu.async_copy(itab_hbm.at[i_idx.at[pl.ds(c * CHUNK, CHUNK)]],
                         i_buf.at[pl.ds(boff, CHUNK)], sem_i)

    issue(0)
    issue(1)

    def chunk_body(c, _):
        boff = (c % 2) * CHUNK
        pltpu.make_async_copy(utab_hbm.at[u_idx.at[pl.ds(0, CHUNK)]],
                              u_buf.at[pl.ds(boff, CHUNK)], sem_u).wait()
        pltpu.make_async_copy(itab_hbm.at[i_idx.at[pl.ds(0, CHUNK)]],
                              i_buf.at[pl.ds(boff, CHUNK)], sem_i).wait()

        def load_row(r):
            return ([u_buf[r, pl.ds(k * L, L)] for k in range(DIM // L)],
                    [i_buf[r, pl.ds(k * L, L)] for k in range(DIM // L)])

        def group_body(g, _):
            row = boff + g * L
            nxt_ld = load_row(row)
            for rr in range(L):
                us, vs = nxt_ld
                if rr + 1 < L:
                    nxt_ld = load_row(row + rr + 1)
                p0 = us[0] * vs[0]
                p1 = us[1] * vs[1]
                p2 = us[2] * vs[2]
                p3 = us[3] * vs[3]
                p4 = us[4] * vs[4]
                p5 = us[5] * vs[5]
                p6 = us[6] * vs[6]
                p7 = us[7] * vs[7]
                s = ((p0 + p1) + (p2 + p3)) + ((p4 + p5) + (p6 + p7))
                accs[rr, pl.ds(0, L)] = s
            t0 = plsc.load_gather(accs, [iota, jnp.zeros((L,), jnp.int32)])
            t1 = plsc.load_gather(accs, [iota, jnp.zeros((L,), jnp.int32) + 1])
            for j in range(2, L, 2):
                t0 = t0 + plsc.load_gather(
                    accs, [iota, jnp.zeros((L,), jnp.int32) + j])
                t1 = t1 + plsc.load_gather(
                    accs, [iota, jnp.zeros((L,), jnp.int32) + j + 1])
            tot = t0 + t1
            o_v[pl.ds(c * CHUNK + g * L, L)] = 1.0 / (1.0 + jnp.exp(-tot))
            return 0

        lax.fori_loop(0, GROUPS, group_body, 0)

        @pl.when(c + 2 < NCHUNK)
        def _():
            pltpu.async_copy(
                utab_hbm.at[u_idx.at[pl.ds((c + 2) * CHUNK, CHUNK)]],
                u_buf.at[pl.ds(boff, CHUNK)], sem_u)
            pltpu.async_copy(
                itab_hbm.at[i_idx.at[pl.ds((c + 2) * CHUNK, CHUNK)]],
                i_buf.at[pl.ds(boff, CHUNK)], sem_i)

        return 0

    lax.fori_loop(0, NCHUNK, chunk_body, 0)

    pltpu.sync_copy(o_v, out_hbm.at[pl.ds(base, BPW)])


@jax.jit
def _gmf(user1d, item1d, user_table, item_table):
    mesh = plsc.VectorSubcoreMesh(core_axis_name="c", subcore_axis_name="s")
    kern = pl.kernel(
        _gmf_body,
        mesh=mesh,
        out_type=jax.ShapeDtypeStruct((BATCH,), jnp.float32),
        compiler_params=pltpu.CompilerParams(needs_layout_passes=False),
        scratch_types=[
            pltpu.VMEM((BPW,), jnp.int32),
            pltpu.VMEM((BPW,), jnp.int32),
            pltpu.VMEM((2 * CHUNK, DIM), jnp.float32),
            pltpu.VMEM((2 * CHUNK, DIM), jnp.float32),
            pltpu.VMEM((BPW,), jnp.float32),
            pltpu.VMEM((L, L + 1), jnp.float32),
            pltpu.SemaphoreType.DMA,
            pltpu.SemaphoreType.DMA,
        ],
    )
    return kern(user1d, item1d, user_table, item_table)


def kernel(user, item, user_table, item_table):
    return _gmf(user.astype(jnp.int32), item.astype(jnp.int32),
                user_table, item_table)
